# baseline (speedup 1.0000x reference)
import numpy as np
import jax
import jax.numpy as jnp
from jax.experimental import pallas as pl
from jax.experimental.pallas import tpu as pltpu

_MP = 7
_MP_PAD = 2
_CK = 3
_SIGMA = 3.0

_k1d = np.exp(-(np.arange(_CK, dtype=np.float32) - _CK // 2) ** 2
              / (2.0 * np.float32(_SIGMA) ** 2)).astype(np.float32)
_k1d = (_k1d / _k1d.sum()).astype(np.float32)

_SROWS = 120
_SLANES = 128
_ROW0 = 4
_NEG = -1.0e30
_LANE0 = 2


def _conv_ops(H, W):
    Ho, Wo = H - 2, W - 2
    L = np.zeros((_SLANES, _SROWS), dtype=np.float32)
    for i in range(Ho):
        for t in range(_CK):
            r = i + t - 1
            if 0 <= r < Ho:
                L[i, r + _ROW0 - _MP_PAD] = _k1d[t]
    R = np.zeros((_SLANES, _SLANES), dtype=np.float32)
    for j in range(Wo):
        for t in range(_CK):
            w = j + t - 1
            if 0 <= w < Wo:
                R[w, j] = _k1d[t]
    return L, R


def _grid2image_block(x_ref, l_ref, r_ref, o_ref, scr_ref):
    bb, depth, H, W = x_ref.shape
    Ho, Wo = H - 2, W - 2
    n = bb * depth

    @pl.when(pl.program_id(0) == 0)
    def _():
        scr_ref[...] = jnp.full((n, _SROWS, _SLANES), _NEG, jnp.float32)

    scr_ref[:, _ROW0:_ROW0 + H, _LANE0:_LANE0 + W] = (
        x_ref[...].reshape(n, H, W))

    sv = scr_ref[...]
    t = jnp.maximum(sv, pltpu.roll(sv, _SROWS - 1, 1))
    t = jnp.maximum(t, pltpu.roll(t, _SROWS - 2, 1))
    t = jnp.maximum(t, pltpu.roll(t, _SROWS - 3, 1))
    s = jnp.maximum(t, pltpu.roll(t, _SLANES - 1, 2))
    s = jnp.maximum(s, pltpu.roll(s, _SLANES - 2, 2))
    p = jnp.maximum(s, pltpu.roll(s, _SLANES - 3, 2))

    pb = p.astype(jnp.bfloat16)
    lm = l_ref[...]
    rm = r_ref[...]
    q = jnp.dot(pb, rm, preferred_element_type=jnp.float32)
    qb = q.astype(jnp.bfloat16)
    for b in range(bb):
        acc = None
        for d in range(depth):
            cv = jnp.dot(lm, qb[b * depth + d],
                         preferred_element_type=jnp.float32)
            acc = cv if acc is None else jnp.maximum(acc, cv)
        img = acc[0:Ho, 0:Wo]
        inv = 1.0 / jnp.max(img)
        out = 1.0 - img * inv
        o_ref[b, 0] = out
        o_ref[b, 1] = out
        o_ref[b, 2] = out


def kernel(x):
    x = x.astype(jnp.float32)
    B, D, H, W = x.shape
    Ho, Wo = H - 2, W - 2

    bb = 8
    while B % bb:
        bb //= 2

    lmat, rmat = _conv_ops(H, W)

    return pl.pallas_call(
        _grid2image_block,
        out_shape=jax.ShapeDtypeStruct((B, 3, Ho, Wo), jnp.float32),
        grid=(B // bb,),
        in_specs=[
            pl.BlockSpec((bb, D, H, W), lambda i: (i, 0, 0, 0)),
            pl.BlockSpec((_SLANES, _SROWS), lambda i: (0, 0)),
            pl.BlockSpec((_SLANES, _SLANES), lambda i: (0, 0)),
        ],
        out_specs=pl.BlockSpec((bb, 3, Ho, Wo), lambda i: (i, 0, 0, 0)),
        scratch_shapes=[pltpu.VMEM((bb * D, _SROWS, _SLANES), jnp.float32)],
        compiler_params=pltpu.CompilerParams(
            dimension_semantics=("parallel",)),
    )(x, jnp.asarray(lmat, jnp.bfloat16), jnp.asarray(rmat, jnp.bfloat16))


# speedup vs baseline: 1.0621x; 1.0621x over previous
import numpy as np
import jax
import jax.numpy as jnp
from jax.experimental import pallas as pl
from jax.experimental.pallas import tpu as pltpu

_MP = 7
_MP_PAD = 2
_CK = 3
_SIGMA = 3.0

_k1d = np.exp(-(np.arange(_CK, dtype=np.float32) - _CK // 2) ** 2
              / (2.0 * np.float32(_SIGMA) ** 2)).astype(np.float32)
_k1d = (_k1d / _k1d.sum()).astype(np.float32)

_SROWS = 120
_SLANES = 128
_ANCH = _MP - 1 - _MP_PAD
_NEG = -1.0e30


def _conv_ops(H, W):
    Ho, Wo = H - 2, W - 2
    L = np.zeros((_SLANES, _SROWS), dtype=np.float32)
    for i in range(Ho):
        for t in range(_CK):
            r = i + t - 1
            if 0 <= r < Ho:
                L[i, r + _ANCH] = _k1d[t]
    R = np.zeros((_SLANES, _SLANES), dtype=np.float32)
    for j in range(Wo):
        for t in range(_CK):
            w = j + t - 1
            if 0 <= w < Wo:
                R[w + _ANCH, j] = _k1d[t]
    return L, R


def _grid2image_block(x_ref, l_ref, r_ref, o_ref, scr_ref):
    bb, depth, H, W = x_ref.shape
    Ho, Wo = H - 2, W - 2
    n = bb * depth

    @pl.when(pl.program_id(0) == 0)
    def _():
        scr_ref[...] = jnp.full((n, _SROWS, _SLANES), _NEG, jnp.float32)

    scr_ref[:, 0:H, 0:W] = x_ref[...].reshape(n, H, W)

    sv = scr_ref[...]
    t = jnp.maximum(sv, pltpu.roll(sv, 1, 1))
    t = jnp.maximum(t, pltpu.roll(t, 2, 1))
    t = jnp.maximum(t, pltpu.roll(t, 3, 1))
    s = jnp.maximum(t, pltpu.roll(t, 1, 2))
    s = jnp.maximum(s, pltpu.roll(s, 2, 2))
    p = jnp.maximum(s, pltpu.roll(s, 3, 2))

    pb = p.astype(jnp.bfloat16)
    lm = l_ref[...]
    rm = r_ref[...]
    for b in range(bb):
        acc = None
        for d in range(depth):
            t2 = jnp.dot(lm, pb[b * depth + d],
                         preferred_element_type=jnp.float32)
            cv = jnp.dot(t2.astype(jnp.bfloat16), rm,
                         preferred_element_type=jnp.float32)
            acc = cv if acc is None else jnp.maximum(acc, cv)
        img = acc[0:Ho, 0:Wo]
        inv = 1.0 / jnp.max(img)
        out = 1.0 - img * inv
        o_ref[b, 0] = out
        o_ref[b, 1] = out
        o_ref[b, 2] = out


def kernel(x):
    x = x.astype(jnp.float32)
    B, D, H, W = x.shape
    Ho, Wo = H - 2, W - 2

    bb = 8
    while B % bb:
        bb //= 2

    lmat, rmat = _conv_ops(H, W)

    return pl.pallas_call(
        _grid2image_block,
        out_shape=jax.ShapeDtypeStruct((B, 3, Ho, Wo), jnp.float32),
        grid=(B // bb,),
        in_specs=[
            pl.BlockSpec((bb, D, H, W), lambda i: (i, 0, 0, 0)),
            pl.BlockSpec((_SLANES, _SROWS), lambda i: (0, 0)),
            pl.BlockSpec((_SLANES, _SLANES), lambda i: (0, 0)),
        ],
        out_specs=pl.BlockSpec((bb, 3, Ho, Wo), lambda i: (i, 0, 0, 0)),
        scratch_shapes=[pltpu.VMEM((bb * D, _SROWS, _SLANES), jnp.float32)],
        compiler_params=pltpu.CompilerParams(
            dimension_semantics=("parallel",)),
    )(x, jnp.asarray(lmat, jnp.bfloat16), jnp.asarray(rmat, jnp.bfloat16))
